# Initial kernel scaffold; baseline (speedup 1.0000x reference)
import jax
import jax.numpy as jnp
from jax.experimental import pallas as pl


def kernel(h, coords, batch, params):
    raise NotImplementedError("write your pallas kernel here")


# speedup vs baseline: 206.8251x; 206.8251x over previous
import functools

import jax
import jax.numpy as jnp
import numpy as np
from jax.experimental import pallas as pl

B, N, HID = 8, 256, 32
NPOOL, NPAD = 128, 258
NC = 264

_W_ORDER = ["em1a", "em1b", "em2_W", "em3_W", "in_W", "out_W",
            "ge1_h1", "ge1_h2", "W_e", "ge2_W", "gn1a", "gn1b",
            "gn2_W", "gc1_W", "gx1_W"]
_B_ORDER = ["em1_b", "em2_b", "em3_b", "bne_w", "bne_b", "in_b",
            "out_b", "ge1_b", "ge2_b", "gn1_b", "gn2_b", "gc1_b",
            "gx1_b", "bnh_w", "bnh_b"]
_WI = {k: i for i, k in enumerate(_W_ORDER)}
_BI = {k: i for i, k in enumerate(_B_ORDER)}


def _constants():
    Pm = np.zeros((NC, N), np.float32)
    Pm[0, 0] = 1.0
    Pm[1:N + 1, :] = np.eye(N, dtype=np.float32)
    Pm[N + 1, N - 1] = 1.0
    Mp = np.zeros((NPOOL, NC), np.float32)
    for i in range(NPOOL):
        Mp[i, i * 2:i * 2 + 3] = np.float32(1.0 / 3.0)
    Cs = np.zeros((3 * NPOOL, NC), np.float32)
    for k in range(3):
        for p in range(NPOOL):
            Cs[k * NPOOL + p, 2 * p + k] = 1.0
    mb = np.zeros((NC, 1), np.float32)
    mb[1:NPAD] = 1.0
    return Pm, Mp, Cs, mb


_PM, _MP, _CS, _MASKB = _constants()


def _silu(x):
    return x * jax.nn.sigmoid(x)


def _egnn_body(h_ref, c_ref, pm_ref, mp_ref, cs_ref, mb_ref,
               w_ref, b_ref, g_ref, ho_ref, co_ref):
    f32 = jnp.float32
    dot = functools.partial(jax.lax.dot, preferred_element_type=f32)

    def W(name):
        i = _WI[name]
        return w_ref[32 * i:32 * (i + 1), :]

    def bias(name):
        return b_ref[_BI[name]:_BI[name] + 1, :]

    def ln(x, wname, bname, eps=1e-5):
        m = jnp.mean(x, axis=-1, keepdims=True)
        v = jnp.mean((x - m) ** 2, axis=-1, keepdims=True)
        return (x - m) / jnp.sqrt(v + eps) * bias(wname) + bias(bname)

    def cross(a, b):
        a0, a1, a2 = a[:, 0:1], a[:, 1:2], a[:, 2:3]
        b0, b1, b2 = b[:, 0:1], b[:, 1:2], b[:, 2:3]
        return jnp.concatenate(
            [a1 * b2 - a2 * b1, a2 * b0 - a0 * b2, a0 * b1 - a1 * b0], axis=1)

    hb = h_ref[...]
    cb = c_ref[...]
    Pm = pm_ref[...]
    Mp = mp_ref[...]

    h3 = dot(Pm, hb)
    c3 = dot(Pm, cb)
    h_pool = dot(Mp, h3)
    c_pool = dot(Mp, c3)

    hh_pool = dot(h_pool, W("in_W")) + bias("in_b")
    A_pool = dot(h_pool, W("em1a"))
    P_pool = dot(hh_pool, W("ge1_h1"))
    hh3 = dot(h3, W("in_W")) + bias("in_b")
    Bc3 = dot(h3, W("em1b")) + bias("em1_b")
    Q3 = dot(hh3, W("ge1_h2"))

    w_r = g_ref[0:1, :]
    gc2 = g_ref[1:33, 0:1]
    gx2 = g_ref[1:33, 1:2]

    def edge_mlp(A_row, P_row, c_row, Bc_col, Q_col, c_col):
        x1 = jnp.maximum(A_row + Bc_col, 0.0)
        x2 = jnp.maximum(dot(x1, W("em2_W")) + bias("em2_b"), 0.0)
        ea = ln(dot(x2, W("em3_W")) + bias("em3_b"), "bne_w", "bne_b")
        cdiff = c_row - c_col
        radial = jnp.sum(cdiff * cdiff, axis=-1, keepdims=True)
        cc = cross(c_row, c_col)
        nrm = jnp.sqrt(jnp.sum(cc * cc, axis=-1, keepdims=True))
        cc = cc / (nrm + 1.0)
        m1 = _silu(P_row + Q_col + dot(radial, w_r) + dot(ea, W("W_e"))
                   + bias("ge1_b"))
        m = _silu(dot(m1, W("ge2_W")) + bias("ge2_b"))
        phi = dot(_silu(dot(m, W("gc1_W")) + bias("gc1_b")), gc2)
        phix = dot(_silu(dot(m, W("gx1_W")) + bias("gx1_b")), gx2)
        trans = cdiff * phi + cc * phix
        return m, trans

    Cs = cs_ref[...]
    aggm = jnp.zeros((NPOOL, HID), f32)
    aggt = jnp.zeros((NPOOL, 3), f32)
    for k in range(3):
        Ck = Cs[k * NPOOL:(k + 1) * NPOOL, :]
        mk, tk = edge_mlp(A_pool, P_pool, c_pool,
                          dot(Ck, Bc3), dot(Ck, Q3), dot(Ck, c3))
        aggm = aggm + mk
        aggt = aggt + tk

    mB, tB = edge_mlp(A_pool[0:1, :], P_pool[0:1, :], c_pool[0:1, :],
                      Bc3, Q3, c3)
    mb = mb_ref[...]
    s_m = jnp.sum(mB * mb, axis=0, keepdims=True)
    s_t = jnp.sum(tB * mb, axis=0, keepdims=True)
    row0 = jax.lax.broadcasted_iota(jnp.int32, (NPOOL, 1), 0) == 0
    aggm = aggm + jnp.where(row0, s_m, 0.0)
    aggt = aggt + jnp.where(row0, s_t, 0.0)

    nup = dot(_silu(dot(hh_pool, W("gn1a")) + dot(aggm, W("gn1b"))
                    + bias("gn1_b")), W("gn2_W")) + bias("gn2_b")
    hh_new = hh_pool + nup
    ho_ref[...] = ln(dot(hh_new, W("out_W")) + bias("out_b"), "bnh_w", "bnh_b")
    co_ref[...] = c_pool + aggt


def kernel(h, coords, batch, params):
    del batch
    p = params
    f32 = jnp.float32
    wstack = jnp.concatenate([
        p["em1_W"][:HID], p["em1_W"][HID:], p["em2_W"], p["em3_W"],
        p["in_W"], p["out_W"], p["ge1_W"][0:HID], p["ge1_W"][HID:2 * HID],
        p["ge1_W"][2 * HID + 1:], p["ge2_W"], p["gn1_W"][:HID],
        p["gn1_W"][HID:], p["gn2_W"], p["gc1_W"], p["gx1_W"],
    ], axis=0).astype(f32)
    bstack = jnp.stack([
        p["em1_b"], p["em2_b"], p["em3_b"], p["bne_w"], p["bne_b"],
        p["in_b"], p["out_b"], p["ge1_b"], p["ge2_b"], p["gn1_b"],
        p["gn2_b"], p["gc1_b"], p["gx1_b"], p["bnh_w"], p["bnh_b"],
        jnp.zeros((HID,), f32),
    ], axis=0).astype(f32)
    gbuf = jnp.zeros((33, 32), f32)
    gbuf = gbuf.at[0:1, :].set(p["ge1_W"][2 * HID:2 * HID + 1])
    gbuf = gbuf.at[1:33, 0:1].set(p["gc2_W"])
    gbuf = gbuf.at[1:33, 1:2].set(p["gx2_W"])

    pm = jnp.asarray(_PM)
    mp = jnp.asarray(_MP)
    cs = jnp.asarray(_CS)
    mb = jnp.asarray(_MASKB)

    const_spec = lambda arr: pl.BlockSpec(arr.shape, lambda b: (0,) * arr.ndim)
    out_h = jax.ShapeDtypeStruct((B * NPOOL, HID), f32)
    out_c = jax.ShapeDtypeStruct((B * NPOOL, 3), f32)
    ho, co = pl.pallas_call(
        _egnn_body,
        grid=(B,),
        in_specs=[
            pl.BlockSpec((N, HID), lambda b: (b, 0)),
            pl.BlockSpec((N, 3), lambda b: (b, 0)),
            const_spec(pm), const_spec(mp), const_spec(cs), const_spec(mb),
            const_spec(wstack), const_spec(bstack), const_spec(gbuf),
        ],
        out_specs=[
            pl.BlockSpec((NPOOL, HID), lambda b: (b, 0)),
            pl.BlockSpec((NPOOL, 3), lambda b: (b, 0)),
        ],
        out_shape=[out_h, out_c],
    )(h.astype(f32), coords.astype(f32), pm, mp, cs, mb, wstack, bstack, gbuf)
    return ho, co


# speedup vs baseline: 381.1143x; 1.8427x over previous
import functools

import jax
import jax.numpy as jnp
import numpy as np
from jax.experimental import pallas as pl

B, N, HID = 8, 256, 32
NPOOL, NPAD = 128, 258
NC = 264
NA = 3 * B * NPOOL
NE = NA + B * NC

_W_ORDER = ["em1a", "em1b", "em2_W", "em3_W", "in_W", "out_W",
            "ge1_h1", "ge1_h2", "W_e", "ge2_W", "gn1a", "gn1b",
            "gn2_W", "gc1_W", "gx1_W"]
_B_ORDER = ["em1_b", "em2_b", "em3_b", "bne_w", "bne_b", "in_b",
            "out_b", "ge1_b", "ge2_b", "gn1_b", "gn2_b", "gc1_b",
            "gx1_b", "bnh_w", "bnh_b"]
_WI = {k: i for i, k in enumerate(_W_ORDER)}
_BI = {k: i for i, k in enumerate(_B_ORDER)}


def _constants():
    SB = np.zeros((B, B * NC), np.float32)
    for b in range(B):
        SB[b, b * NC + 1:b * NC + NPAD] = 1.0
    E2 = np.zeros((B * NPOOL, B), np.float32)
    for b in range(B):
        E2[b * NPOOL, b] = 1.0
    CS = np.zeros((3 * NPOOL, NC), np.float32)
    for k in range(3):
        for p in range(NPOOL):
            CS[k * NPOOL + p, 2 * p + k] = 1.0
    return SB, E2, CS


_SB, _E2, _CS = _constants()


def _silu(x):
    return x * jax.nn.sigmoid(x)


def _egnn_body(h_ref, c_ref, sb_ref, e2_ref, cs_ref, w_ref, b_ref, g_ref,
               ho_ref, co_ref):
    f32 = jnp.float32
    dot = functools.partial(jax.lax.dot, preferred_element_type=f32)

    def W(name):
        i = _WI[name]
        return w_ref[32 * i:32 * (i + 1), :]

    def bias(name):
        return b_ref[_BI[name]:_BI[name] + 1, :]

    def ln(x, wname, bname, eps=1e-5):
        m = jnp.mean(x, axis=-1, keepdims=True)
        v = jnp.mean((x - m) ** 2, axis=-1, keepdims=True)
        return (x - m) / jnp.sqrt(v + eps) * bias(wname) + bias(bname)

    def cross(a, b):
        a0, a1, a2 = a[:, 0:1], a[:, 1:2], a[:, 2:3]
        b0, b1, b2 = b[:, 0:1], b[:, 1:2], b[:, 2:3]
        return jnp.concatenate(
            [a1 * b2 - a2 * b1, a2 * b0 - a0 * b2, a0 * b1 - a1 * b0], axis=1)

    CS = cs_ref[...]
    ch_h = [[], [], []]
    ch_c = [[], [], []]
    colB_h, colB_c = [], []
    zeros_h = jnp.zeros((NC - NPAD, HID), f32)
    zeros_c = jnp.zeros((NC - NPAD, 3), f32)
    for b in range(B):
        hb = h_ref[b * N:(b + 1) * N, :]
        cb = c_ref[b * N:(b + 1) * N, :]
        h3 = jnp.concatenate([hb[0:1, :], hb, hb[N - 1:N, :], zeros_h],
                             axis=0)
        c3 = jnp.concatenate([cb[0:1, :], cb, cb[N - 1:N, :], zeros_c],
                             axis=0)
        sel_h = dot(CS, h3)
        sel_c = dot(CS, c3)
        for k in range(3):
            ch_h[k].append(sel_h[k * NPOOL:(k + 1) * NPOOL, :])
            ch_c[k].append(sel_c[k * NPOOL:(k + 1) * NPOOL, :])
        colB_h.append(h3)
        colB_c.append(c3)

    ch_h = [jnp.concatenate(x, axis=0) for x in ch_h]
    ch_c = [jnp.concatenate(x, axis=0) for x in ch_c]
    h_pool = (ch_h[0] + ch_h[1] + ch_h[2]) * f32(1.0 / 3.0)
    c_pool = (ch_c[0] + ch_c[1] + ch_c[2]) * f32(1.0 / 3.0)

    colh = jnp.concatenate(ch_h + colB_h, axis=0)
    colc = jnp.concatenate(ch_c + colB_c, axis=0)

    hh_pool = dot(h_pool, W("in_W")) + bias("in_b")
    A_pool = dot(h_pool, W("em1a"))
    P_pool = dot(hh_pool, W("ge1_h1"))
    W_q = dot(W("in_W"), W("ge1_h2"))
    b_q = dot(bias("in_b"), W("ge1_h2"))
    Bc_col = dot(colh, W("em1b")) + bias("em1_b")
    Q_col = dot(colh, W_q) + b_q

    rowsA = [A_pool, P_pool, c_pool]
    rowB_A, rowB_P, rowB_c = [], [], []
    for b in range(B):
        r = b * NPOOL
        rowB_A.append(jnp.broadcast_to(A_pool[r:r + 1, :], (NC, HID)))
        rowB_P.append(jnp.broadcast_to(P_pool[r:r + 1, :], (NC, HID)))
        rowB_c.append(jnp.broadcast_to(c_pool[r:r + 1, :], (NC, 3)))
    A_row = jnp.concatenate([A_pool] * 3 + rowB_A, axis=0)
    P_row = jnp.concatenate([P_pool] * 3 + rowB_P, axis=0)
    c_row = jnp.concatenate([c_pool] * 3 + rowB_c, axis=0)

    w_r = g_ref[0:1, :]
    gc2 = g_ref[1:33, 0:1]
    gx2 = g_ref[1:33, 1:2]

    x1 = jnp.maximum(A_row + Bc_col, 0.0)
    x2 = jnp.maximum(dot(x1, W("em2_W")) + bias("em2_b"), 0.0)
    ea = ln(dot(x2, W("em3_W")) + bias("em3_b"), "bne_w", "bne_b")
    cdiff = c_row - colc
    radial = jnp.sum(cdiff * cdiff, axis=-1, keepdims=True)
    cc = cross(c_row, colc)
    nrm = jnp.sqrt(jnp.sum(cc * cc, axis=-1, keepdims=True))
    cc = cc / (nrm + 1.0)
    m1 = _silu(P_row + Q_col + dot(radial, w_r) + dot(ea, W("W_e"))
               + bias("ge1_b"))
    m = _silu(dot(m1, W("ge2_W")) + bias("ge2_b"))
    phi = dot(_silu(dot(m, W("gc1_W")) + bias("gc1_b")), gc2)
    phix = dot(_silu(dot(m, W("gx1_W")) + bias("gx1_b")), gx2)
    trans = cdiff * phi + cc * phix

    NP = B * NPOOL
    aggm = m[0:NP, :] + m[NP:2 * NP, :] + m[2 * NP:3 * NP, :]
    aggt = trans[0:NP, :] + trans[NP:2 * NP, :] + trans[2 * NP:3 * NP, :]
    SB = sb_ref[...]
    E2 = e2_ref[...]
    aggm = aggm + dot(E2, dot(SB, m[NA:, :]))
    aggt = aggt + dot(E2, dot(SB, trans[NA:, :]))

    nup = dot(_silu(dot(hh_pool, W("gn1a")) + dot(aggm, W("gn1b"))
                    + bias("gn1_b")), W("gn2_W")) + bias("gn2_b")
    hh_new = hh_pool + nup
    ho_ref[...] = ln(dot(hh_new, W("out_W")) + bias("out_b"), "bnh_w", "bnh_b")
    co_ref[...] = c_pool + aggt


def kernel(h, coords, batch, params):
    del batch
    p = params
    f32 = jnp.float32
    wstack = jnp.concatenate([
        p["em1_W"][:HID], p["em1_W"][HID:], p["em2_W"], p["em3_W"],
        p["in_W"], p["out_W"], p["ge1_W"][0:HID], p["ge1_W"][HID:2 * HID],
        p["ge1_W"][2 * HID + 1:], p["ge2_W"], p["gn1_W"][:HID],
        p["gn1_W"][HID:], p["gn2_W"], p["gc1_W"], p["gx1_W"],
    ], axis=0).astype(f32)
    bstack = jnp.stack([
        p["em1_b"], p["em2_b"], p["em3_b"], p["bne_w"], p["bne_b"],
        p["in_b"], p["out_b"], p["ge1_b"], p["ge2_b"], p["gn1_b"],
        p["gn2_b"], p["gc1_b"], p["gx1_b"], p["bnh_w"], p["bnh_b"],
        jnp.zeros((HID,), f32),
    ], axis=0).astype(f32)
    gbuf = jnp.zeros((33, 32), f32)
    gbuf = gbuf.at[0:1, :].set(p["ge1_W"][2 * HID:2 * HID + 1])
    gbuf = gbuf.at[1:33, 0:1].set(p["gc2_W"])
    gbuf = gbuf.at[1:33, 1:2].set(p["gx2_W"])

    sb = jnp.asarray(_SB)
    e2 = jnp.asarray(_E2)
    cs = jnp.asarray(_CS)

    out_h = jax.ShapeDtypeStruct((B * NPOOL, HID), f32)
    out_c = jax.ShapeDtypeStruct((B * NPOOL, 3), f32)
    ho, co = pl.pallas_call(
        _egnn_body,
        out_shape=[out_h, out_c],
    )(h.astype(f32), coords.astype(f32), sb, e2, cs, wstack, bstack, gbuf)
    return ho, co
